# baseline (speedup 1.0000x reference)
import functools

import jax
import jax.numpy as jnp
from jax import lax
from jax.experimental import pallas as pl
from jax.experimental.pallas import tpu as pltpu
from jax.experimental.pallas import tpu_sc as plsc

_CHUNK = 64
_NBUF = 6


@jax.jit
def _gather_rows(idx, table3):
    C, V, D = table3.shape
    B = idx.shape[0]

    info = plsc.get_sparse_core_info()
    num_workers = info.num_cores * info.num_subcores
    b_per_w = B // num_workers
    n_chunks = b_per_w // _CHUNK

    mesh = plsc.VectorSubcoreMesh(core_axis_name="c", subcore_axis_name="s")

    @functools.partial(
        pl.kernel,
        mesh=mesh,
        out_type=jax.ShapeDtypeStruct((C, B, D), jnp.float32),
        scratch_types=[
            pltpu.VMEM((b_per_w,), jnp.int32),
            pltpu.VMEM((_NBUF, _CHUNK, D), jnp.float32),
            pltpu.SemaphoreType.DMA,
            pltpu.SemaphoreType.DMA,
        ],
    )
    def k(idx_hbm, table_hbm, out_hbm, idx_v, rows_v, gsem, ssem):
        wid = lax.axis_index("s") * info.num_cores + lax.axis_index("c")
        base = wid * b_per_w
        pltpu.sync_copy(idx_hbm.at[pl.ds(base, b_per_w)], idx_v)

        tasks = [(p, g) for p in range(C) for g in range(n_chunks)]
        T = len(tasks)

        def gather(t):
            p, g = tasks[t]
            return pltpu.async_copy(
                table_hbm.at[p].at[idx_v.at[pl.ds(g * _CHUNK, _CHUNK)]],
                rows_v.at[t % _NBUF],
                gsem,
            )

        def store(t):
            p, g = tasks[t]
            return pltpu.async_copy(
                rows_v.at[t % _NBUF],
                out_hbm.at[p].at[pl.ds(base + g * _CHUNK, _CHUNK)],
                ssem,
            )

        gathers = {t: gather(t) for t in range(_NBUF - 1)}
        stores = {}
        for t in range(T):
            gathers[t].wait()
            stores[t] = store(t)
            if t + _NBUF - 1 < T:
                if t - 1 >= 0:
                    stores[t - 1].wait()
                gathers[t + _NBUF - 1] = gather(t + _NBUF - 1)
        for t in range(max(0, T - _NBUF), T):
            stores[t].wait()

    return k(idx, table3)


def kernel(class_number, multi_vector_embedding):
    table3 = jnp.transpose(multi_vector_embedding, (2, 0, 1))
    out3 = _gather_rows(class_number.astype(jnp.int32), table3)
    return jnp.transpose(out3, (1, 2, 0))


# speedup vs baseline: 1.0183x; 1.0183x over previous
import functools

import jax
import jax.numpy as jnp
from jax import lax
from jax.experimental import pallas as pl
from jax.experimental.pallas import tpu as pltpu
from jax.experimental.pallas import tpu_sc as plsc

_CHUNK = 64
_NBUF = 4


@jax.jit
def _gather_rows(idx, table3):
    C, V, D = table3.shape
    B = idx.shape[0]

    info = plsc.get_sparse_core_info()
    num_workers = info.num_cores * info.num_subcores
    b_per_w = B // num_workers
    n_chunks = b_per_w // _CHUNK

    mesh = plsc.VectorSubcoreMesh(core_axis_name="c", subcore_axis_name="s")

    @functools.partial(
        pl.kernel,
        mesh=mesh,
        out_type=jax.ShapeDtypeStruct((C, B, D), jnp.float32),
        scratch_types=[
            pltpu.VMEM((b_per_w,), jnp.int32),
            pltpu.VMEM((_NBUF, _CHUNK, D), jnp.float32),
            pltpu.VMEM_SHARED((info.num_subcores, 2, _CHUNK, D), jnp.float32),
            pltpu.SemaphoreType.DMA,
            pltpu.SemaphoreType.DMA,
        ],
    )
    def k(idx_hbm, table_hbm, out_hbm, idx_v, rows_v, stage_sh, gsem, ssem):
        sid = lax.axis_index("s")
        wid = sid * info.num_cores + lax.axis_index("c")
        base = wid * b_per_w
        pltpu.sync_copy(idx_hbm.at[pl.ds(base, b_per_w)], idx_v)

        tasks = [(p, g) for p in range(C) for g in range(n_chunks)]
        T = len(tasks)

        def gather(t):
            p, g = tasks[t]
            return pltpu.async_copy(
                table_hbm.at[p].at[idx_v.at[pl.ds(g * _CHUNK, _CHUNK)]],
                rows_v.at[t % _NBUF],
                gsem,
            )

        gathers = {t: gather(t) for t in range(_NBUF - 1)}
        stores = {}
        for t in range(T):
            gathers[t].wait()
            if t - 2 >= 0:
                stores[t - 2].wait()
            pltpu.sync_copy(rows_v.at[t % _NBUF], stage_sh.at[sid, t % 2])
            if t + _NBUF - 1 < T:
                gathers[t + _NBUF - 1] = gather(t + _NBUF - 1)
            p, g = tasks[t]
            stores[t] = pltpu.async_copy(
                stage_sh.at[sid, t % 2],
                out_hbm.at[p].at[pl.ds(base + g * _CHUNK, _CHUNK)],
                ssem,
            )
        stores[T - 2].wait()
        stores[T - 1].wait()

    return k(idx, table3)


def kernel(class_number, multi_vector_embedding):
    table3 = jnp.transpose(multi_vector_embedding, (2, 0, 1))
    out3 = _gather_rows(class_number.astype(jnp.int32), table3)
    return jnp.transpose(out3, (1, 2, 0))
